# baseline (speedup 1.0000x reference)
import functools

import jax
import jax.numpy as jnp
from jax import lax
from jax.experimental import pallas as pl
from jax.experimental.pallas import tpu as pltpu
from jax.experimental.pallas import tpu_sc as plsc

D_MODEL = 64
LANES = 16
NUM_CORES = 2
NUM_SUBCORES = 16
NUM_WORKERS = NUM_CORES * NUM_SUBCORES
GROUP = 128
SCALE = 8.0


@functools.partial(jax.jit, static_argnums=(2, 3))
def _embed(x_grouped, table, per_worker, n_groups):
    mesh = plsc.VectorSubcoreMesh(core_axis_name="c", subcore_axis_name="s")
    total_rows = NUM_WORKERS * per_worker

    @functools.partial(
        pl.kernel,
        mesh=mesh,
        out_type=jax.ShapeDtypeStruct((total_rows, D_MODEL), jnp.float32),
        scratch_types=[
            pltpu.VMEM((n_groups, GROUP), jnp.int32),
            pltpu.VMEM((GROUP, D_MODEL), jnp.float32),
            pltpu.SemaphoreType.DMA,
        ],
        compiler_params=pltpu.CompilerParams(use_tc_tiling_on_sc=False),
    )
    def k(x_hbm, table_hbm, out_hbm, idx_v, rows_v, gsem):
        wid = lax.axis_index("s") * NUM_CORES + lax.axis_index("c")
        base = wid * per_worker
        pltpu.sync_copy(x_hbm.at[wid], idx_v)

        def group_body(g, carry):
            pltpu.async_copy(table_hbm.at[idx_v.at[g]], rows_v, gsem).wait()

            def scale_body(i, c):
                for j in range(D_MODEL // LANES):
                    sl = pl.ds(j * LANES, LANES)
                    rows_v[i, sl] = rows_v[i, sl] * SCALE
                return c

            lax.fori_loop(0, GROUP, scale_body, 0, unroll=4)
            pltpu.sync_copy(
                rows_v, out_hbm.at[pl.ds(base + g * GROUP, GROUP)]
            )
            return carry

        lax.fori_loop(0, n_groups, group_body, 0)

    return k(x_grouped, table)


def kernel(x, embedding_weight):
    batch, seq = x.shape
    total = batch * seq
    per_worker = total // NUM_WORKERS
    n_groups = per_worker // GROUP
    x_grouped = x.reshape(NUM_WORKERS, n_groups, GROUP).astype(jnp.int32)
    out = _embed(x_grouped, embedding_weight, per_worker, n_groups)
    return out.reshape(batch, seq, D_MODEL)
